# baseline (speedup 1.0000x reference)
import math

import jax
import jax.numpy as jnp
from jax.experimental import pallas as pl
from jax.experimental.pallas import tpu as pltpu
from jax.sharding import PartitionSpec as P

try:
    _shard_map = jax.shard_map
except AttributeError:
    from jax.experimental.shard_map import shard_map as _shard_map

N = 512
H = 64
NR = 60
NB = 4
NM = 32
CUTOFF = 5.0
TI = 32
NCH = N // TI
KG = NR + TI + 1

_HIGHEST = jax.lax.Precision.HIGHEST

_WIDTH = CUTOFF / NR
_INV = 1.0 / (2.0 * _WIDTH * _WIDTH)
_SCALE = math.sqrt(_INV * math.log2(math.e))


def _silu_from_half(r):
    return r + r * jnp.tanh(r)


def _prep_body(cs_ref, an_ref, posr_ref, posh_ref, emb_ref,
               x_ref, d_ref, m_ref, cnt_ref):
    f32 = jnp.float32
    nh = posh_ref.shape[1]

    an = jnp.clip(an_ref[...], 0, 99)
    onehot = (jax.lax.broadcasted_iota(jnp.int32, (100, N), 0) == an).astype(f32)
    x_ref[...] = jnp.dot(emb_ref[...], onehot, preferred_element_type=f32)

    posr = posr_ref[...]
    posh = posh_ref[...]
    cross = jax.lax.dot_general(posr, posh, (((1,), (0,)), ((), ())),
                                precision=_HIGHEST,
                                preferred_element_type=f32)
    p2row = jnp.sum(posr * posr, axis=1, keepdims=True)
    p2col = jnp.sum(posh * posh, axis=0, keepdims=True)
    dsq = jnp.maximum(p2row + p2col - 2.0 * cross, 0.0)
    row = jax.lax.broadcasted_iota(jnp.int32, (N, nh), 0)
    col = jax.lax.broadcasted_iota(jnp.int32, (N, nh), 1) + cs_ref[0, 0]
    maskf = ((row != col) & (dsq < CUTOFF * CUTOFF)).astype(f32)
    d_ref[...] = jnp.sqrt(dsq) * _SCALE
    m_ref[...] = maskf
    cnt_ref[...] = jnp.dot(jnp.ones((1, N), f32), maskf,
                           preferred_element_type=f32)


def _block_body(xf_ref, xh_ref, d_ref, m_ref, cnt_ref,
                w1x, w1r, b1, w2, b2, u1, ub1, u2, ub2,
                out_ref, g_ref):
    f32 = jnp.float32
    nh = d_ref.shape[1]
    e = TI * nh

    lane = jax.lax.broadcasted_iota(jnp.int32, (1, e), 1)
    Rm = (jax.lax.broadcasted_iota(jnp.int32, (TI, e), 0) == lane // nh).astype(f32)
    g_ref[NR:NR + TI, :] = Rm
    g_ref[NR + TI:KG, :] = jnp.ones((1, e), f32)

    centers_s = (jax.lax.broadcasted_iota(jnp.int32, (NR, 1), 0).astype(f32)
                 * (CUTOFF / (NR - 1) * _SCALE))
    ei_row = jax.lax.broadcasted_iota(jnp.int32, (N, TI), 0)
    ei_col = jax.lax.broadcasted_iota(jnp.int32, (N, TI), 1)

    xw1h = jnp.dot(w1x[...] * 0.5, xf_ref[...], preferred_element_type=f32)
    w1rh, b1h = w1r[...] * 0.5, b1[...] * 0.5

    def chunk(c, hsum):
        for t in range(TI):
            drow = d_ref[pl.ds(c * TI + t, 1), :]
            y = drow - centers_s
            g_ref[0:NR, t * nh:(t + 1) * nh] = jnp.exp2(-(y * y))
        ec = (ei_row == c * TI + ei_col).astype(f32)
        xc = jnp.dot(xw1h, ec, preferred_element_type=f32)
        wcat = jnp.concatenate([w1rh, xc, b1h], axis=1)
        r = jnp.dot(wcat, g_ref[...], preferred_element_type=f32)
        for t in range(TI):
            mrow = m_ref[pl.ds(c * TI + t, 1), :]
            h_t = _silu_from_half(r[:, t * nh:(t + 1) * nh])
            hsum = hsum + h_t * mrow
        return hsum

    hsum = jax.lax.fori_loop(0, NCH, chunk, jnp.zeros((H, nh), f32))

    aggr = (jnp.dot(w2[...], hsum, preferred_element_type=f32)
            + b2[...] * cnt_ref[...])
    u = jnp.concatenate([xh_ref[...], aggr], axis=0)
    hu = _silu_from_half(
        jnp.dot(u1[...] * 0.5, u, preferred_element_type=f32) + ub1[...] * 0.5)
    out_ref[...] = (xh_ref[...] + jnp.dot(u2[...], hu, preferred_element_type=f32)
                    + ub2[...])


def _pool_body(x_ref, batc_ref, ow1_ref, ob1_ref, ow2_ref, ob2_ref, y_ref):
    f32 = jnp.float32
    sel = (batc_ref[...] == jax.lax.broadcasted_iota(jnp.int32, (1, NM), 1)).astype(f32)
    mol = jnp.dot(x_ref[...], sel, preferred_element_type=f32)
    cntm = jnp.sum(sel, axis=0, keepdims=True)
    mol = mol / jnp.clip(cntm, 1.0, None)
    ho = _silu_from_half(
        jnp.dot(ow1_ref[...] * 0.5, mol, preferred_element_type=f32)
        + ob1_ref[...] * 0.5)
    y_ref[...] = jnp.dot(ow2_ref[...], ho, preferred_element_type=f32) + ob2_ref[...]


def kernel(atomic_numbers, positions, batch, emb, blocks, out_w1, out_b1, out_w2, out_b2):
    f32 = jnp.float32
    nsh = 2 if jax.device_count() >= 2 else 1
    nh = N // nsh

    anT = jnp.asarray(atomic_numbers, jnp.int32).reshape(1, N)
    posr = jnp.asarray(positions, f32)
    posT = posr.T
    batc = jnp.asarray(batch, jnp.int32).reshape(N, 1)
    embT = jnp.asarray(emb, f32).T
    wflat = []
    for blk in blocks:
        wflat += [
            blk['msg_w1'][:H].T, blk['msg_w1'][H:].T, blk['msg_b1'].reshape(H, 1),
            blk['msg_w2'].T, blk['msg_b2'].reshape(H, 1),
            blk['upd_w1'].T, blk['upd_b1'].reshape(H, 1),
            blk['upd_w2'].T, blk['upd_b2'].reshape(H, 1),
        ]
    ows = (out_w1.T, out_b1.reshape(H // 2, 1), out_w2.T, out_b2.reshape(1, 1))

    def shard_fn(anT, posr, posT, batc, embT, wflat, ows):
        idx = jax.lax.axis_index('d')
        colstart = jnp.asarray(idx * nh, jnp.int32).reshape(1, 1)
        posh = jax.lax.dynamic_slice(posT, (0, idx * nh), (3, nh))
        xT, dh, mh, cnth = pl.pallas_call(
            _prep_body,
            out_shape=(jax.ShapeDtypeStruct((H, N), f32),
                       jax.ShapeDtypeStruct((N, nh), f32),
                       jax.ShapeDtypeStruct((N, nh), f32),
                       jax.ShapeDtypeStruct((1, nh), f32)),
        )(colstart, anT, posr, posh, embT)
        xh = jax.lax.dynamic_slice(xT, (0, idx * nh), (H, nh))
        for b in range(NB):
            xh = pl.pallas_call(
                _block_body,
                out_shape=jax.ShapeDtypeStruct((H, nh), f32),
                scratch_shapes=[pltpu.VMEM((KG, TI * nh), f32)],
            )(xT, xh, dh, mh, cnth, *wflat[9 * b:9 * (b + 1)])
            xT = jax.lax.all_gather(xh, 'd', axis=1, tiled=True)
        y = pl.pallas_call(
            _pool_body,
            out_shape=jax.ShapeDtypeStruct((1, NM), f32),
        )(xT, batc, *ows)
        return y

    mesh = jax.make_mesh((nsh,), ('d',))
    yall = _shard_map(
        shard_fn, mesh=mesh,
        in_specs=(P(), P(), P(), P(), P(), P(), P()),
        out_specs=P(), check_vma=False,
    )(anT, posr, posT, batc, embT, tuple(wflat), ows)
    return yall.reshape(NM, 1)


# speedup vs baseline: 4.7218x; 4.7218x over previous
import math

import jax
import jax.numpy as jnp
from jax.experimental import pallas as pl
from jax.experimental.pallas import tpu as pltpu

N = 512
H = 64
NR = 60
NB = 4
NM = 32
CUTOFF = 5.0
TI = 32
NCH = N // TI
E = TI * N
KG = NR + TI + 1

_HIGHEST = jax.lax.Precision.HIGHEST

_WIDTH = CUTOFF / NR
_INV = 1.0 / (2.0 * _WIDTH * _WIDTH)
_SCALE = math.sqrt(_INV * math.log2(math.e))


def _silu_from_half(r):
    return r + r * jnp.tanh(r)


def _centers_s():
    return (jax.lax.broadcasted_iota(jnp.int32, (NR, 1), 0).astype(jnp.float32)
            * (CUTOFF / (NR - 1) * _SCALE))


def _rbf_body(posr_ref, pos_ref, rbf_ref, m_ref, cnt_ref, d_ref):
    f32 = jnp.float32
    c = pl.program_id(0)

    @pl.when(c == 0)
    def _init():
        posr = posr_ref[...]
        pos = pos_ref[...]
        cross = jax.lax.dot_general(posr, pos, (((1,), (0,)), ((), ())),
                                    precision=_HIGHEST,
                                    preferred_element_type=f32)
        p2row = jnp.sum(posr * posr, axis=1, keepdims=True)
        p2col = jnp.sum(pos * pos, axis=0, keepdims=True)
        dsq = jnp.maximum(p2row + p2col - 2.0 * cross, 0.0)
        row = jax.lax.broadcasted_iota(jnp.int32, (N, N), 0)
        col = jax.lax.broadcasted_iota(jnp.int32, (N, N), 1)
        maskf = ((row != col) & (dsq < CUTOFF * CUTOFF)).astype(f32)
        d_ref[...] = jnp.sqrt(dsq) * _SCALE
        m_ref[...] = maskf
        cnt_ref[...] = jnp.dot(jnp.ones((1, N), f32), maskf,
                               preferred_element_type=f32)

    centers = _centers_s()
    for t in range(TI):
        drow = d_ref[pl.ds(c * TI + t, 1), :]
        y = drow - centers
        rbf_ref[0:NR, t * N:(t + 1) * N] = jnp.exp2(-(y * y))


def _main_body(rbf_ref, m_ref, cnt_ref, an_ref, emb_ref, batc_ref,
               w1x_ref, w1r_ref, b1_ref, w2_ref, b2_ref,
               u1_ref, ub1_ref, u2_ref, ub2_ref,
               ow1_ref, ob1_ref, ow2_ref, ob2_ref,
               y_ref, g_ref, x_ref, xw1_ref, hs_ref):
    f32 = jnp.float32
    b = pl.program_id(0)
    c = pl.program_id(1)

    @pl.when((b == 0) & (c == 0))
    def _first():
        an = jnp.clip(an_ref[...], 0, 99)
        onehot = (jax.lax.broadcasted_iota(jnp.int32, (100, N), 0) == an).astype(f32)
        x_ref[...] = jnp.dot(emb_ref[...], onehot, preferred_element_type=f32)
        lane = jax.lax.broadcasted_iota(jnp.int32, (1, E), 1)
        g_ref[NR:NR + TI, :] = (
            jax.lax.broadcasted_iota(jnp.int32, (TI, E), 0) == lane // N
        ).astype(f32)
        g_ref[NR + TI:KG, :] = jnp.ones((1, E), f32)

    @pl.when(c == 0)
    def _block_start():
        xw1_ref[...] = jnp.dot(w1x_ref[0] * 0.5, x_ref[...],
                               preferred_element_type=f32)
        hs_ref[...] = jnp.zeros((H, N), f32)

    g_ref[0:NR, :] = rbf_ref[...]
    ec = (jax.lax.broadcasted_iota(jnp.int32, (N, TI), 0)
          == c * TI + jax.lax.broadcasted_iota(jnp.int32, (N, TI), 1)).astype(f32)
    xc = jnp.dot(xw1_ref[...], ec, preferred_element_type=f32)
    wcat = jnp.concatenate([w1r_ref[0] * 0.5, xc, b1_ref[0] * 0.5], axis=1)
    r = jnp.dot(wcat, g_ref[...], preferred_element_type=f32)
    hsum = hs_ref[...]
    for t in range(TI):
        mrow = m_ref[pl.ds(c * TI + t, 1), :]
        h_t = _silu_from_half(r[:, t * N:(t + 1) * N])
        hsum = hsum + h_t * mrow
    hs_ref[...] = hsum

    @pl.when(c == NCH - 1)
    def _block_end():
        aggr = (jnp.dot(w2_ref[0], hs_ref[...], preferred_element_type=f32)
                + b2_ref[0] * cnt_ref[...])
        u = jnp.concatenate([x_ref[...], aggr], axis=0)
        hu = _silu_from_half(
            jnp.dot(u1_ref[0] * 0.5, u, preferred_element_type=f32)
            + ub1_ref[0] * 0.5)
        x_ref[...] = (x_ref[...]
                      + jnp.dot(u2_ref[0], hu, preferred_element_type=f32)
                      + ub2_ref[0])

    @pl.when((b == NB - 1) & (c == NCH - 1))
    def _final():
        sel = (batc_ref[...] == jax.lax.broadcasted_iota(jnp.int32, (1, NM), 1)
               ).astype(f32)
        mol = jnp.dot(x_ref[...], sel, preferred_element_type=f32)
        cntm = jnp.sum(sel, axis=0, keepdims=True)
        mol = mol / jnp.clip(cntm, 1.0, None)
        ho = _silu_from_half(
            jnp.dot(ow1_ref[...] * 0.5, mol, preferred_element_type=f32)
            + ob1_ref[...] * 0.5)
        y_ref[...] = (jnp.dot(ow2_ref[...], ho, preferred_element_type=f32)
                      + ob2_ref[...])


def kernel(atomic_numbers, positions, batch, emb, blocks, out_w1, out_b1, out_w2, out_b2):
    f32 = jnp.float32
    anT = jnp.asarray(atomic_numbers, jnp.int32).reshape(1, N)
    posr = jnp.asarray(positions, f32)
    posT = posr.T
    batc = jnp.asarray(batch, jnp.int32).reshape(N, 1)
    embT = jnp.asarray(emb, f32).T

    rbf, mgrid, cnt = pl.pallas_call(
        _rbf_body,
        grid=(NCH,),
        in_specs=[pl.BlockSpec((N, 3), lambda c: (0, 0)),
                  pl.BlockSpec((3, N), lambda c: (0, 0))],
        out_specs=(pl.BlockSpec((NR, E), lambda c: (0, c)),
                   pl.BlockSpec((N, N), lambda c: (0, 0)),
                   pl.BlockSpec((1, N), lambda c: (0, 0))),
        out_shape=(jax.ShapeDtypeStruct((NR, NCH * E), f32),
                   jax.ShapeDtypeStruct((N, N), f32),
                   jax.ShapeDtypeStruct((1, N), f32)),
        scratch_shapes=[pltpu.VMEM((N, N), f32)],
    )(posr, posT)

    stk = {k: jnp.stack([blk[k] for blk in blocks]) for k in blocks[0]}
    w1x = jnp.swapaxes(stk['msg_w1'][:, :H, :], 1, 2)
    w1r = jnp.swapaxes(stk['msg_w1'][:, H:, :], 1, 2)
    b1 = stk['msg_b1'].reshape(NB, H, 1)
    w2 = jnp.swapaxes(stk['msg_w2'], 1, 2)
    b2 = stk['msg_b2'].reshape(NB, H, 1)
    u1 = jnp.swapaxes(stk['upd_w1'], 1, 2)
    ub1 = stk['upd_b1'].reshape(NB, H, 1)
    u2 = jnp.swapaxes(stk['upd_w2'], 1, 2)
    ub2 = stk['upd_b2'].reshape(NB, H, 1)

    def bw(shape):
        return pl.BlockSpec((1,) + shape, lambda b, c: (b, 0, 0))

    def const2(shape):
        return pl.BlockSpec(shape, lambda b, c: (0, 0))

    yT = pl.pallas_call(
        _main_body,
        grid=(NB, NCH),
        in_specs=[pl.BlockSpec((NR, E), lambda b, c: (0, c)),
                  const2((N, N)), const2((1, N)), const2((1, N)),
                  const2((H, 100)), const2((N, 1)),
                  bw((H, H)), bw((H, NR)), bw((H, 1)), bw((H, H)), bw((H, 1)),
                  bw((H, 2 * H)), bw((H, 1)), bw((H, H)), bw((H, 1)),
                  const2((NM, H)), const2((NM, 1)),
                  const2((1, NM)), const2((1, 1))],
        out_specs=pl.BlockSpec((1, NM), lambda b, c: (0, 0)),
        out_shape=jax.ShapeDtypeStruct((1, NM), f32),
        scratch_shapes=[pltpu.VMEM((KG, E), f32),
                        pltpu.VMEM((H, N), f32),
                        pltpu.VMEM((H, N), f32),
                        pltpu.VMEM((H, N), f32)],
    )(rbf, mgrid, cnt, anT, embT, batc,
      w1x, w1r, b1, w2, b2, u1, ub1, u2, ub2,
      out_w1.T, out_b1.reshape(H // 2, 1), out_w2.T, out_b2.reshape(1, 1))
    return yT.reshape(NM, 1)


# speedup vs baseline: 5.3716x; 1.1376x over previous
import math

import jax
import jax.numpy as jnp
from jax.experimental import pallas as pl
from jax.experimental.pallas import tpu as pltpu

N = 512
H = 64
NR = 60
NB = 4
NM = 32
CUTOFF = 5.0
TI = 32
NCH = N // TI
E = TI * N
KG = NR + TI + 1

_HIGHEST = jax.lax.Precision.HIGHEST


def _silu_from_half(r):
    return r + r * jnp.tanh(r)


def _body(an_ref, posr_ref, pos_ref, batc_ref, emb_ref, *rest):
    f32 = jnp.float32
    wrefs = rest[:9 * NB]
    (ow1_ref, ob1_ref, ow2_ref, ob2_ref, y_ref,
     g_ref, d_ref, m_ref) = rest[9 * NB:]

    an = jnp.clip(an_ref[...], 0, 99)
    onehot = (jax.lax.broadcasted_iota(jnp.int32, (100, N), 0) == an).astype(f32)
    xT = jnp.dot(emb_ref[...], onehot, preferred_element_type=f32)

    pos = pos_ref[...]
    posr = posr_ref[...]

    width = CUTOFF / NR
    inv = 1.0 / (2.0 * width * width)
    scale = math.sqrt(inv * math.log2(math.e))
    centers_s = (jax.lax.broadcasted_iota(jnp.int32, (NR, 1), 0).astype(f32)
                 * (CUTOFF / (NR - 1) * scale))

    cross = jax.lax.dot_general(posr, pos, (((1,), (0,)), ((), ())),
                                precision=_HIGHEST,
                                preferred_element_type=f32)
    p2row = jnp.sum(posr * posr, axis=1, keepdims=True)
    p2col = jnp.sum(pos * pos, axis=0, keepdims=True)
    dsq = jnp.maximum(p2row + p2col - 2.0 * cross, 0.0)
    fr_row = jax.lax.broadcasted_iota(jnp.int32, (N, N), 0)
    fr_col = jax.lax.broadcasted_iota(jnp.int32, (N, N), 1)
    keep = (fr_row != fr_col) & (dsq < CUTOFF * CUTOFF)
    maskf = keep.astype(f32)
    d_ref[...] = jnp.where(keep, jnp.sqrt(dsq) * scale, 1e4)
    m_ref[...] = maskf
    cnt = jnp.dot(jnp.ones((1, N), f32), maskf, preferred_element_type=f32)

    g_ref[NR:KG, :] = jnp.zeros((TI + 1, E), f32)

    ei_row = jax.lax.broadcasted_iota(jnp.int32, (N, TI), 0)
    ei_col = jax.lax.broadcasted_iota(jnp.int32, (N, TI), 1)

    for b in range(NB):
        w1x, w1r, b1, w2, b2, u1, ub1, u2, ub2 = wrefs[9 * b:9 * (b + 1)]
        xw1h = jnp.dot(w1x[...] * 0.5, xT, preferred_element_type=f32)
        w1rh, b1h = w1r[...] * 0.5, b1[...] * 0.5
        u1h, ub1h = u1[...] * 0.5, ub1[...] * 0.5

        def chunk(c, hsum, xw1h=xw1h, w1rh=w1rh, b1h=b1h):
            for t in range(TI):
                drow = d_ref[pl.ds(c * TI + t, 1), :]
                mrow = m_ref[pl.ds(c * TI + t, 1), :]
                y = drow - centers_s
                g_ref[0:NR, t * N:(t + 1) * N] = jnp.exp2(-(y * y))
                g_ref[NR + t:NR + t + 1, t * N:(t + 1) * N] = mrow
                g_ref[KG - 1:KG, t * N:(t + 1) * N] = mrow
            ec = (ei_row == c * TI + ei_col).astype(f32)
            xc = jnp.dot(xw1h, ec, preferred_element_type=f32)
            wcat = jnp.concatenate([w1rh, xc, b1h], axis=1)
            r = jnp.dot(wcat, g_ref[...], preferred_element_type=f32)
            for t in range(TI):
                hsum = hsum + _silu_from_half(r[:, t * N:(t + 1) * N])
            return hsum

        hsum = jax.lax.fori_loop(0, NCH, chunk, jnp.zeros((H, N), f32))

        aggr = jnp.dot(w2[...], hsum, preferred_element_type=f32) + b2[...] * cnt
        u = jnp.concatenate([xT, aggr], axis=0)
        hu = _silu_from_half(
            jnp.dot(u1h, u, preferred_element_type=f32) + ub1h)
        xT = xT + jnp.dot(u2[...], hu, preferred_element_type=f32) + ub2[...]

    sel = (batc_ref[...] == jax.lax.broadcasted_iota(jnp.int32, (1, NM), 1)).astype(f32)
    mol = jnp.dot(xT, sel, preferred_element_type=f32)
    cntm = jnp.sum(sel, axis=0, keepdims=True)
    mol = mol / jnp.clip(cntm, 1.0, None)
    ho = _silu_from_half(
        jnp.dot(ow1_ref[...] * 0.5, mol, preferred_element_type=f32)
        + ob1_ref[...] * 0.5)
    y_ref[...] = jnp.dot(ow2_ref[...], ho, preferred_element_type=f32) + ob2_ref[...]


def kernel(atomic_numbers, positions, batch, emb, blocks, out_w1, out_b1, out_w2, out_b2):
    f32 = jnp.float32
    anT = jnp.asarray(atomic_numbers, jnp.int32).reshape(1, N)
    posr = jnp.asarray(positions, f32)
    posT = posr.T
    batc = jnp.asarray(batch, jnp.int32).reshape(N, 1)
    embT = jnp.asarray(emb, f32).T
    wflat = []
    for blk in blocks:
        wflat += [
            blk['msg_w1'][:H].T, blk['msg_w1'][H:].T, blk['msg_b1'].reshape(H, 1),
            blk['msg_w2'].T, blk['msg_b2'].reshape(H, 1),
            blk['upd_w1'].T, blk['upd_b1'].reshape(H, 1),
            blk['upd_w2'].T, blk['upd_b2'].reshape(H, 1),
        ]
    yT = pl.pallas_call(
        _body,
        out_shape=jax.ShapeDtypeStruct((1, NM), f32),
        scratch_shapes=[pltpu.VMEM((KG, E), f32),
                        pltpu.VMEM((N, N), f32),
                        pltpu.VMEM((N, N), f32)],
    )(anT, posr, posT, batc, embT, *wflat,
      out_w1.T, out_b1.reshape(H // 2, 1), out_w2.T, out_b2.reshape(1, 1))
    return yT.reshape(NM, 1)


# speedup vs baseline: 5.6598x; 1.0536x over previous
import math

import jax
import jax.numpy as jnp
from jax.experimental import pallas as pl
from jax.experimental.pallas import tpu as pltpu

N = 512
H = 64
NR = 60
NB = 4
NM = 32
CUTOFF = 5.0
TI = 32
NCH = N // TI
E = TI * N
KG = NR + TI

_HIGHEST = jax.lax.Precision.HIGHEST


def _silu_from_half(r):
    return r + r * jnp.tanh(r)


def _body(an_ref, posr_ref, pos_ref, batc_ref, emb_ref, *rest):
    f32 = jnp.float32
    wrefs = rest[:9 * NB]
    (ow1_ref, ob1_ref, ow2_ref, ob2_ref, y_ref,
     g_ref, d_ref, m_ref) = rest[9 * NB:]

    an = jnp.clip(an_ref[...], 0, 99)
    onehot = (jax.lax.broadcasted_iota(jnp.int32, (100, N), 0) == an).astype(f32)
    xT = jnp.dot(emb_ref[...], onehot, preferred_element_type=f32)

    pos = pos_ref[...]
    posr = posr_ref[...]

    width = CUTOFF / NR
    inv = 1.0 / (2.0 * width * width)
    scale = math.sqrt(inv * math.log2(math.e))
    centers_s = (jax.lax.broadcasted_iota(jnp.int32, (NR, 1), 0).astype(f32)
                 * (CUTOFF / (NR - 1) * scale))

    cross = jax.lax.dot_general(posr, pos, (((1,), (0,)), ((), ())),
                                precision=_HIGHEST,
                                preferred_element_type=f32)
    p2row = jnp.sum(posr * posr, axis=1, keepdims=True)
    p2col = jnp.sum(pos * pos, axis=0, keepdims=True)
    dsq = jnp.maximum(p2row + p2col - 2.0 * cross, 0.0)
    fr_row = jax.lax.broadcasted_iota(jnp.int32, (N, N), 0)
    fr_col = jax.lax.broadcasted_iota(jnp.int32, (N, N), 1)
    keep = (fr_row != fr_col) & (dsq < CUTOFF * CUTOFF)
    maskf = keep.astype(f32)
    d_ref[...] = jnp.where(keep, jnp.sqrt(dsq) * scale, 1e4)
    m_ref[...] = maskf
    cnt = jnp.dot(jnp.ones((1, N), f32), maskf, preferred_element_type=f32)

    g_ref[NR:KG, :] = jnp.zeros((TI, E), f32)

    ei_row = jax.lax.broadcasted_iota(jnp.int32, (N, TI), 0)
    ei_col = jax.lax.broadcasted_iota(jnp.int32, (N, TI), 1)

    for b in range(NB):
        w1x, w1r, b1, w2, b2, u1, ub1, u2, ub2 = wrefs[9 * b:9 * (b + 1)]
        xw1h = jnp.dot(w1x[...] * 0.5, xT, preferred_element_type=f32)
        w1rh, b1h = w1r[...] * 0.5, b1[...] * 0.5
        u1h, ub1h = u1[...] * 0.5, ub1[...] * 0.5

        def chunk(c, hsum, xw1h=xw1h, w1rh=w1rh, b1h=b1h):
            for t in range(TI):
                drow = d_ref[pl.ds(c * TI + t, 1), :]
                mrow = m_ref[pl.ds(c * TI + t, 1), :]
                y = drow - centers_s
                g_ref[0:NR, t * N:(t + 1) * N] = jnp.exp2(-(y * y))
                g_ref[NR + t:NR + t + 1, t * N:(t + 1) * N] = mrow
            ec = (ei_row == c * TI + ei_col).astype(f32)
            xc = jnp.dot(xw1h, ec, preferred_element_type=f32) + b1h
            wcat = jnp.concatenate([w1rh, xc], axis=1)
            r = jnp.dot(wcat, g_ref[...], preferred_element_type=f32)
            for t in range(TI):
                hsum = hsum + _silu_from_half(r[:, t * N:(t + 1) * N])
            return hsum

        hsum = jax.lax.fori_loop(0, NCH, chunk, jnp.zeros((H, N), f32),
                                 unroll=2)

        aggr = jnp.dot(w2[...], hsum, preferred_element_type=f32) + b2[...] * cnt
        u = jnp.concatenate([xT, aggr], axis=0)
        hu = _silu_from_half(
            jnp.dot(u1h, u, preferred_element_type=f32) + ub1h)
        xT = xT + jnp.dot(u2[...], hu, preferred_element_type=f32) + ub2[...]

    sel = (batc_ref[...] == jax.lax.broadcasted_iota(jnp.int32, (1, NM), 1)).astype(f32)
    mol = jnp.dot(xT, sel, preferred_element_type=f32)
    cntm = jnp.sum(sel, axis=0, keepdims=True)
    mol = mol / jnp.clip(cntm, 1.0, None)
    ho = _silu_from_half(
        jnp.dot(ow1_ref[...] * 0.5, mol, preferred_element_type=f32)
        + ob1_ref[...] * 0.5)
    y_ref[...] = jnp.dot(ow2_ref[...], ho, preferred_element_type=f32) + ob2_ref[...]


def kernel(atomic_numbers, positions, batch, emb, blocks, out_w1, out_b1, out_w2, out_b2):
    f32 = jnp.float32
    anT = jnp.asarray(atomic_numbers, jnp.int32).reshape(1, N)
    posr = jnp.asarray(positions, f32)
    posT = posr.T
    batc = jnp.asarray(batch, jnp.int32).reshape(N, 1)
    embT = jnp.asarray(emb, f32).T
    wflat = []
    for blk in blocks:
        wflat += [
            blk['msg_w1'][:H].T, blk['msg_w1'][H:].T, blk['msg_b1'].reshape(H, 1),
            blk['msg_w2'].T, blk['msg_b2'].reshape(H, 1),
            blk['upd_w1'].T, blk['upd_b1'].reshape(H, 1),
            blk['upd_w2'].T, blk['upd_b2'].reshape(H, 1),
        ]
    yT = pl.pallas_call(
        _body,
        out_shape=jax.ShapeDtypeStruct((1, NM), f32),
        scratch_shapes=[pltpu.VMEM((KG, E), f32),
                        pltpu.VMEM((N, N), f32),
                        pltpu.VMEM((N, N), f32)],
    )(anT, posr, posT, batc, embT, *wflat,
      out_w1.T, out_b1.reshape(H // 2, 1), out_w2.T, out_b2.reshape(1, 1))
    return yT.reshape(NM, 1)
